# baseline (speedup 1.0000x reference)
import jax
import jax.numpy as jnp
from jax import lax
from jax.experimental import pallas as pl
from jax.experimental.pallas import tpu as pltpu
from jax.experimental.pallas import tpu_sc as plsc

N_COLS = 14
N_ROWS = 131072
N_WORKERS = 32
LANES_PER_WORKER = N_ROWS // N_WORKERS
CHUNK_LENS = [512, 1536, 1536, 512]
CHUNK_OFFS = [0, 512, 2048, 3584]
N_CHUNKS = len(CHUNK_LENS)
MAX_CHUNK = max(CHUNK_LENS)

PARENT = [0, 1] + [c // 2 - 1 for c in range(2, N_COLS)]


def _tree_softmax_body(in_hbm, out_hbm, in0, in1, out0, out1,
                       in_sems, out_sems):
    in_bufs = [in0, in1]
    out_bufs = [out0, out1]
    wid = lax.axis_index("s") * 2 + lax.axis_index("c")
    base = pl.multiple_of(wid * LANES_PER_WORKER, 512)

    def chunk_slice(i):
        off = pl.multiple_of(base + CHUNK_OFFS[i], 512)
        return pl.ds(off, CHUNK_LENS[i])

    def copy_in(i, slot):
        return pltpu.make_async_copy(
            in_hbm.at[:, chunk_slice(i)],
            in_bufs[slot].at[:, pl.ds(0, CHUNK_LENS[i])], in_sems.at[slot])

    def copy_out(i, slot):
        return pltpu.make_async_copy(
            out_bufs[slot].at[:, pl.ds(0, CHUNK_LENS[i])],
            out_hbm.at[:, chunk_slice(i)], out_sems.at[slot])

    def compute(i, slot):
        src = in_bufs[slot]
        dst = out_bufs[slot]

        @plsc.parallel_loop(0, CHUNK_LENS[i] // 16, unroll=2)
        def group(g):
            sl = pl.ds(g * 16, 16)
            x = [src[c, sl] for c in range(N_COLS)]
            f = [None] * N_COLS
            for i in range(N_COLS // 2):
                a, b = x[2 * i], x[2 * i + 1]
                inv = 1.0 / (1.0 + jnp.exp(b - a))
                f[2 * i] = inv
                f[2 * i + 1] = 1.0 - inv
            for c in range(2, N_COLS):
                f[c] = f[c] * f[PARENT[c]]
            for c in range(N_COLS):
                dst[c, sl] = f[c]

    copy_in(0, 0).start()
    for i in range(N_CHUNKS):
        slot = i % 2
        if i + 1 < N_CHUNKS:
            copy_in(i + 1, 1 - slot).start()
        copy_in(i, slot).wait()
        if i >= 2:
            copy_out(i - 2, slot).wait()
        compute(i, slot)
        copy_out(i, slot).start()
    copy_out(N_CHUNKS - 2, N_CHUNKS % 2).wait()
    copy_out(N_CHUNKS - 1, (N_CHUNKS - 1) % 2).wait()


@jax.jit
def kernel(input):
    mesh = plsc.VectorSubcoreMesh(core_axis_name="c", subcore_axis_name="s")
    out_t = pl.kernel(
        _tree_softmax_body,
        out_type=jax.ShapeDtypeStruct((N_COLS, N_ROWS), jnp.float32),
        mesh=mesh,
        compiler_params=pltpu.CompilerParams(
            needs_layout_passes=False, use_tc_tiling_on_sc=True),
        scratch_types=[
            pltpu.VMEM((N_COLS, MAX_CHUNK), jnp.float32),
            pltpu.VMEM((N_COLS, MAX_CHUNK), jnp.float32),
            pltpu.VMEM((N_COLS, MAX_CHUNK), jnp.float32),
            pltpu.VMEM((N_COLS, MAX_CHUNK), jnp.float32),
            pltpu.SemaphoreType.DMA((2,)),
            pltpu.SemaphoreType.DMA((2,)),
        ],
    )(input.T)
    return out_t.T
